# baseline (speedup 1.0000x reference)
import jax
import jax.numpy as jnp
from jax import lax
from jax.experimental import pallas as pl
from jax.experimental.pallas import tpu as pltpu

_E = 64
_D = 2048
_ALPHA = 0.01
_T = 2048
_C = 256


def _chunk(x_ref, w_ref, b_ref, mask_ref, wout_ref, acc_mask, acc_prob,
           acc_z, j):
    r0 = j * _C
    l = lax.dot_general(
        x_ref[pl.ds(r0, _C), :], w_ref[...],
        dimension_numbers=(((1,), (1,)), ((), ())),
        preferred_element_type=jnp.float32,
    ) + b_ref[...]

    m = jnp.max(l, axis=-1, keepdims=True)
    e = jnp.exp(l - m)
    s = jnp.sum(e, axis=-1, keepdims=True)
    p = e * (1.0 / s)
    lse = m + jnp.log(s)

    sel1 = e >= 1.0
    m2 = jnp.max(jnp.where(sel1, -1.0, e), axis=-1, keepdims=True)
    mask = jnp.where(sel1 | (e >= m2), 1.0, 0.0).astype(jnp.float32)

    mask_ref[pl.ds(r0, _C), :] = mask
    wout_ref[pl.ds(r0, _C), :] = p * mask

    acc_mask[...] += jnp.sum(mask, axis=0, keepdims=True)
    acc_prob[...] += jnp.sum(p, axis=0, keepdims=True)
    acc_z[...] += jnp.sum(lse * lse).reshape(1, 1)


def kernel(inputs, W, b):
    orig_dtype = inputs.dtype
    x = inputs.astype(jnp.float32).reshape(-1, _D)
    n_tok = x.shape[0]
    n_blocks = n_tok // _T
    b2 = b.reshape(1, _E).astype(jnp.float32)

    def body(x_ref, w_ref, b_ref, mask_ref, wout_ref, loss_ref,
             acc_mask, acc_prob, acc_z):
        i = pl.program_id(0)

        @pl.when(i == 0)
        def _init():
            acc_mask[...] = jnp.zeros_like(acc_mask)
            acc_prob[...] = jnp.zeros_like(acc_prob)
            acc_z[...] = jnp.zeros_like(acc_z)

        def step(j, carry):
            _chunk(x_ref, w_ref, b_ref, mask_ref, wout_ref,
                   acc_mask, acc_prob, acc_z, j)
            return carry

        lax.fori_loop(0, _T // _C, step, 0)

        @pl.when(i == n_blocks - 1)
        def _final():
            inv_n = 1.0 / n_tok
            aux = _ALPHA * _E * jnp.sum(
                (acc_mask[...] * inv_n) * (acc_prob[...] * inv_n))
            loss_ref[...] = aux.reshape(1, 1) + acc_z[...] * inv_n

    mask, wout, loss = pl.pallas_call(
        body,
        grid=(n_blocks,),
        in_specs=[
            pl.BlockSpec((_T, _D), lambda i: (i, 0)),
            pl.BlockSpec((_E, _D), lambda i: (0, 0)),
            pl.BlockSpec((1, _E), lambda i: (0, 0)),
        ],
        out_specs=[
            pl.BlockSpec((_T, _E), lambda i: (i, 0)),
            pl.BlockSpec((_T, _E), lambda i: (i, 0)),
            pl.BlockSpec((1, 1), lambda i: (0, 0)),
        ],
        out_shape=[
            jax.ShapeDtypeStruct((n_tok, _E), jnp.float32),
            jax.ShapeDtypeStruct((n_tok, _E), jnp.float32),
            jax.ShapeDtypeStruct((1, 1), jnp.float32),
        ],
        scratch_shapes=[
            pltpu.VMEM((1, _E), jnp.float32),
            pltpu.VMEM((1, _E), jnp.float32),
            pltpu.VMEM((1, 1), jnp.float32),
        ],
    )(x, W, b2)

    return (mask, wout.astype(orig_dtype), loss[0, 0])


# speedup vs baseline: 1.1390x; 1.1390x over previous
import jax
import jax.numpy as jnp
from jax import lax
from jax.experimental import pallas as pl
from jax.experimental.pallas import tpu as pltpu

_E = 64
_D = 2048
_ALPHA = 0.01
_T = 2048
_C = 512


def _chunk(x_ref, w_ref, b_ref, mask_ref, wout_ref, acc_mask, acc_prob,
           acc_z, j):
    r0 = j * _C
    l = lax.dot_general(
        x_ref[pl.ds(r0, _C), :], w_ref[...],
        dimension_numbers=(((1,), (1,)), ((), ())),
        preferred_element_type=jnp.float32,
    ) + b_ref[...]

    m = jnp.max(l, axis=-1, keepdims=True)
    e = jnp.exp(l - m)
    s = jnp.sum(e, axis=-1, keepdims=True)
    p = e * (1.0 / s)
    lse = m + jnp.log(s)

    sel1 = e >= 1.0
    m2 = jnp.max(jnp.where(sel1, -1.0, e), axis=-1, keepdims=True)
    mask = jnp.where(sel1 | (e >= m2), 1.0, 0.0).astype(jnp.float32)

    mask_ref[pl.ds(r0, _C), :] = mask
    wout_ref[pl.ds(r0, _C), :] = p * mask

    acc_mask[...] += jnp.sum(mask, axis=0, keepdims=True)
    acc_prob[...] += jnp.sum(p, axis=0, keepdims=True)
    acc_z[...] += jnp.sum(lse * lse).reshape(1, 1)


def kernel(inputs, W, b):
    orig_dtype = inputs.dtype
    x = inputs.astype(jnp.float32).reshape(-1, _D)
    n_tok = x.shape[0]
    n_blocks = n_tok // _T
    b2 = b.reshape(1, _E).astype(jnp.float32)

    def body(x_ref, w_ref, b_ref, mask_ref, wout_ref, loss_ref,
             acc_mask, acc_prob, acc_z):
        i = pl.program_id(0)

        @pl.when(i == 0)
        def _init():
            acc_mask[...] = jnp.zeros_like(acc_mask)
            acc_prob[...] = jnp.zeros_like(acc_prob)
            acc_z[...] = jnp.zeros_like(acc_z)

        for j in range(_T // _C):
            _chunk(x_ref, w_ref, b_ref, mask_ref, wout_ref,
                   acc_mask, acc_prob, acc_z, j)

        @pl.when(i == n_blocks - 1)
        def _final():
            inv_n = 1.0 / n_tok
            aux = _ALPHA * _E * jnp.sum(
                (acc_mask[...] * inv_n) * (acc_prob[...] * inv_n))
            loss_ref[...] = aux.reshape(1, 1) + acc_z[...] * inv_n

    mask, wout, loss = pl.pallas_call(
        body,
        grid=(n_blocks,),
        in_specs=[
            pl.BlockSpec((_T, _D), lambda i: (i, 0)),
            pl.BlockSpec((_E, _D), lambda i: (0, 0)),
            pl.BlockSpec((1, _E), lambda i: (0, 0)),
        ],
        out_specs=[
            pl.BlockSpec((_T, _E), lambda i: (i, 0)),
            pl.BlockSpec((_T, _E), lambda i: (i, 0)),
            pl.BlockSpec((1, 1), lambda i: (0, 0)),
        ],
        out_shape=[
            jax.ShapeDtypeStruct((n_tok, _E), jnp.float32),
            jax.ShapeDtypeStruct((n_tok, _E), jnp.float32),
            jax.ShapeDtypeStruct((1, 1), jnp.float32),
        ],
        scratch_shapes=[
            pltpu.VMEM((1, _E), jnp.float32),
            pltpu.VMEM((1, _E), jnp.float32),
            pltpu.VMEM((1, 1), jnp.float32),
        ],
    )(x, W, b2)

    return (mask, wout.astype(orig_dtype), loss[0, 0])
